# Initial kernel scaffold; baseline (speedup 1.0000x reference)
import jax
import jax.numpy as jnp
from jax.experimental import pallas as pl


def kernel(dense_scalar_ids, dense_seq_ids, sparse_scalar_ids, sparse_seq_ids, dense_table, sparse_table, W, b):
    raise NotImplementedError("write your pallas kernel here")


# speedup vs baseline: 1.8217x; 1.8217x over previous
import functools

import jax
import jax.numpy as jnp
from jax import lax
from jax.experimental import pallas as pl
from jax.experimental.pallas import tpu as pltpu
from jax.experimental.pallas import tpu_sc as plsc

NC = 2
NS = 16
NW = NC * NS
L = 16

B = 16384
E = 16
N_DNS = 10
N_DSEQ = 50
N_SNS = 26
N_SSEQ = 20

C = 64
B_PER_W = B // NW
N_CHUNKS = B_PER_W // C

K_DNS = C * N_DNS // 128
K_DSEQ = C * N_DSEQ // 128
K_SNS = C * N_SNS // 128
K_SSEQ = C * N_SSEQ // 128
K_TOTAL = K_DNS + K_DSEQ + K_SNS + K_SSEQ


def _sc_body(dense_table, sparse_table, dns_ids, dseq_ids, sns_ids, sseq_ids,
             dns_out, dmean_out, sns_out, smean_out,
             dns_idx, dseq_idx, sns_idx, sseq_idx,
             dns_buf, dseq_buf, sns_buf, sseq_buf,
             dmean_s, smean_s, sem):
  wid = lax.axis_index("s") * NC + lax.axis_index("c")
  base0 = wid * B_PER_W

  def chunk_body(cidx, _):
    base = base0 + cidx * C

    pltpu.sync_copy(dns_ids.at[pl.ds(pl.multiple_of(base * N_DNS, 128), C * N_DNS)], dns_idx)
    pltpu.sync_copy(dseq_ids.at[pl.ds(pl.multiple_of(base * N_DSEQ, 128), C * N_DSEQ)], dseq_idx)
    pltpu.sync_copy(sns_ids.at[pl.ds(pl.multiple_of(base * N_SNS, 128), C * N_SNS)], sns_idx)
    pltpu.sync_copy(sseq_ids.at[pl.ds(pl.multiple_of(base * N_SSEQ, 128), C * N_SSEQ)], sseq_idx)

    def fire(k, table, idx, buf):
      def go(j, _):
        off = pl.multiple_of(j * 128, 128)
        pltpu.async_copy(table.at[idx.at[pl.ds(off, 128)]],
                         buf.at[pl.ds(off, 128), :], sem)
        return 0
      lax.fori_loop(0, k, go, 0)

    fire(K_DNS, dense_table, dns_idx, dns_buf)
    fire(K_DSEQ, dense_table, dseq_idx, dseq_buf)
    fire(K_SNS, sparse_table, sns_idx, sns_buf)
    fire(K_SSEQ, sparse_table, sseq_idx, sseq_buf)

    def drain(j, _):
      pltpu.make_async_copy(dense_table.at[dns_idx.at[pl.ds(0, 128)]],
                            dns_buf.at[pl.ds(0, 128), :], sem).wait()
      return 0
    lax.fori_loop(0, K_TOTAL, drain, 0)

    def row_body(i, _):
      db = i * N_DSEQ
      acc = dseq_buf[db]
      for t in range(1, N_DSEQ):
        acc = acc + dseq_buf[db + t]
      dmean_s[i] = acc * (1.0 / N_DSEQ)
      sb = i * N_SSEQ
      acc2 = sseq_buf[sb]
      for t in range(1, N_SSEQ):
        acc2 = acc2 + sseq_buf[sb + t]
      smean_s[i] = acc2 * (1.0 / N_SSEQ)
      return 0
    lax.fori_loop(0, C, row_body, 0)

    pltpu.sync_copy(dns_buf, dns_out.at[pl.ds(pl.multiple_of(base * N_DNS, 128), C * N_DNS), :])
    pltpu.sync_copy(sns_buf, sns_out.at[pl.ds(pl.multiple_of(base * N_SNS, 128), C * N_SNS), :])
    pltpu.sync_copy(dmean_s, dmean_out.at[pl.ds(pl.multiple_of(base, 64), C), :])
    pltpu.sync_copy(smean_s, smean_out.at[pl.ds(pl.multiple_of(base, 64), C), :])
    return 0

  lax.fori_loop(0, N_CHUNKS, chunk_body, 0)


_sc_gather = functools.partial(
    pl.kernel,
    out_type=(
        jax.ShapeDtypeStruct((B * N_DNS, E), jnp.float32),
        jax.ShapeDtypeStruct((B, E), jnp.float32),
        jax.ShapeDtypeStruct((B * N_SNS, E), jnp.float32),
        jax.ShapeDtypeStruct((B, E), jnp.float32),
    ),
    mesh=plsc.VectorSubcoreMesh(core_axis_name="c", subcore_axis_name="s"),
    compiler_params=pltpu.CompilerParams(use_tc_tiling_on_sc=False),
    scratch_types=[
        pltpu.VMEM((C * N_DNS,), jnp.int32),
        pltpu.VMEM((C * N_DSEQ,), jnp.int32),
        pltpu.VMEM((C * N_SNS,), jnp.int32),
        pltpu.VMEM((C * N_SSEQ,), jnp.int32),
        pltpu.VMEM((C * N_DNS, E), jnp.float32),
        pltpu.VMEM((C * N_DSEQ, E), jnp.float32),
        pltpu.VMEM((C * N_SNS, E), jnp.float32),
        pltpu.VMEM((C * N_SSEQ, E), jnp.float32),
        pltpu.VMEM((C, E), jnp.float32),
        pltpu.VMEM((C, E), jnp.float32),
        pltpu.SemaphoreType.DMA,
    ],
)(_sc_body)


BM = 512


def _tc_body(dns_ref, dm_ref, sns_ref, sm_ref, w_ref, b_ref, out_ref):
  w = w_ref[...]
  acc = jnp.dot(dns_ref[...], w[0:160], preferred_element_type=jnp.float32)
  acc += jnp.dot(dm_ref[...], w[160:176], preferred_element_type=jnp.float32)
  acc += jnp.dot(sns_ref[...], w[176:592], preferred_element_type=jnp.float32)
  acc += jnp.dot(sm_ref[...], w[592:608], preferred_element_type=jnp.float32)
  acc += b_ref[0:1, :]
  out_ref[...] = jnp.maximum(acc, 0.0)


def _tc_matmul(dns, dmean, sns, smean, w, b8):
  grid = (B // BM,)
  return pl.pallas_call(
      _tc_body,
      grid=grid,
      in_specs=[
          pl.BlockSpec((BM, N_DNS * E), lambda i: (i, 0)),
          pl.BlockSpec((BM, E), lambda i: (i, 0)),
          pl.BlockSpec((BM, N_SNS * E), lambda i: (i, 0)),
          pl.BlockSpec((BM, E), lambda i: (i, 0)),
          pl.BlockSpec((608, 128), lambda i: (0, 0)),
          pl.BlockSpec((8, 128), lambda i: (0, 0)),
      ],
      out_specs=pl.BlockSpec((BM, 128), lambda i: (i, 0)),
      out_shape=jax.ShapeDtypeStruct((B, 128), jnp.float32),
  )(dns, dmean, sns, smean, w, b8)


def kernel(dense_scalar_ids, dense_seq_ids, sparse_scalar_ids, sparse_seq_ids,
           dense_table, sparse_table, W, b):
  dns_ids = dense_scalar_ids.reshape(-1)
  dseq_ids = dense_seq_ids.reshape(-1)
  sns_ids = sparse_scalar_ids.reshape(-1)
  sseq_ids = sparse_seq_ids.reshape(-1)
  dns_rows, dmean, sns_rows, smean = _sc_gather(
      dense_table, sparse_table, dns_ids, dseq_ids, sns_ids, sseq_ids)
  dns = dns_rows.reshape(B, N_DNS * E)
  sns = sns_rows.reshape(B, N_SNS * E)
  b8 = jnp.broadcast_to(b, (8, 128))
  return _tc_matmul(dns, dmean, sns, smean, W, b8)


# speedup vs baseline: 1.8287x; 1.0039x over previous
import functools

import jax
import jax.numpy as jnp
from jax import lax
from jax.experimental import pallas as pl
from jax.experimental.pallas import tpu as pltpu
from jax.experimental.pallas import tpu_sc as plsc

NC = 2
NS = 16
NW = NC * NS

B = 16384
E = 16
N_DNS = 10
N_DSEQ = 50
N_SNS = 26
N_SSEQ = 20
G = 5

C = 32
B_PER_W = B // NW
N_CHUNKS = B_PER_W // C


def _sc_body(dense_table, sparse_table, dns_ids, dseq_ids, sns_ids, sseq_ids,
             out0, out1, out2, out3, out4,
             dns_idx, dseq_idx, sns_idx, sseq_idx,
             dns_buf, dseq_buf, sns_buf, sseq_buf,
             a0, a1, a2, a3, a4, sem):
  wid = lax.axis_index("s") * NC + lax.axis_index("c")
  base0 = wid * B_PER_W
  zeros = jnp.zeros((16,), jnp.float32)

  def chunk_body(cidx, _):
    base = base0 + cidx * C

    pltpu.sync_copy(dns_ids.at[pl.ds(pl.multiple_of(base * N_DNS, 128), C * N_DNS)], dns_idx)
    pltpu.sync_copy(dseq_ids.at[pl.ds(pl.multiple_of(base * N_DSEQ, 128), C * N_DSEQ)], dseq_idx)
    pltpu.sync_copy(sns_ids.at[pl.ds(pl.multiple_of(base * N_SNS, 128), C * N_SNS)], sns_idx)
    pltpu.sync_copy(sseq_ids.at[pl.ds(pl.multiple_of(base * N_SSEQ, 128), C * N_SSEQ)], sseq_idx)

    def fire(n, table, idx, buf):
      full, rem = n // 128, n % 128
      def go(j, _):
        off = pl.multiple_of(j * 128, 128)
        pltpu.async_copy(table.at[idx.at[pl.ds(off, 128)]],
                         buf.at[pl.ds(off, 128), :], sem)
        return 0
      lax.fori_loop(0, full, go, 0)
      if rem:
        off = full * 128
        pltpu.async_copy(table.at[idx.at[pl.ds(off, rem)]],
                         buf.at[pl.ds(off, rem), :], sem)

    fire(C * N_DNS, dense_table, dns_idx, dns_buf)
    fire(C * N_DSEQ, dense_table, dseq_idx, dseq_buf)
    fire(C * N_SNS, sparse_table, sns_idx, sns_buf)
    fire(C * N_SSEQ, sparse_table, sseq_idx, sseq_buf)

    def drain(n, idx, buf):
      full, rem = n // 128, n % 128
      def dr(j, _):
        pltpu.make_async_copy(dense_table.at[idx.at[pl.ds(0, 128)]],
                              buf.at[pl.ds(0, 128), :], sem).wait()
        return 0
      lax.fori_loop(0, full, dr, 0)
      if rem:
        pltpu.make_async_copy(dense_table.at[idx.at[pl.ds(0, rem)]],
                              buf.at[pl.ds(0, rem), :], sem).wait()

    drain(C * N_DNS, dns_idx, dns_buf)
    drain(C * N_DSEQ, dseq_idx, dseq_buf)
    drain(C * N_SNS, sns_idx, sns_buf)
    drain(C * N_SSEQ, sseq_idx, sseq_buf)

    def row_body(i, _):
      db = i * N_DSEQ
      acc = dseq_buf[db]
      for t in range(1, N_DSEQ):
        acc = acc + dseq_buf[db + t]
      dmean = acc * (1.0 / N_DSEQ)
      sb = i * N_SSEQ
      acc2 = sseq_buf[sb]
      for t in range(1, N_SSEQ):
        acc2 = acc2 + sseq_buf[sb + t]
      smean = acc2 * (1.0 / N_SSEQ)

      dn = i * N_DNS
      sn = i * N_SNS
      for j in range(8):
        a0[i, pl.ds(16 * j, 16)] = dns_buf[dn + j]
      a1[i, pl.ds(0, 16)] = dns_buf[dn + 8]
      a1[i, pl.ds(16, 16)] = dns_buf[dn + 9]
      a1[i, pl.ds(32, 16)] = dmean
      for j in range(5):
        a1[i, pl.ds(48 + 16 * j, 16)] = sns_buf[sn + j]
      for j in range(8):
        a2[i, pl.ds(16 * j, 16)] = sns_buf[sn + 5 + j]
      for j in range(8):
        a3[i, pl.ds(16 * j, 16)] = sns_buf[sn + 13 + j]
      for j in range(5):
        a4[i, pl.ds(16 * j, 16)] = sns_buf[sn + 21 + j]
      a4[i, pl.ds(80, 16)] = smean
      a4[i, pl.ds(96, 16)] = zeros
      a4[i, pl.ds(112, 16)] = zeros
      return 0
    lax.fori_loop(0, C, row_body, 0)

    off = pl.multiple_of(base, C)
    pltpu.sync_copy(a0, out0.at[pl.ds(off, C), :])
    pltpu.sync_copy(a1, out1.at[pl.ds(off, C), :])
    pltpu.sync_copy(a2, out2.at[pl.ds(off, C), :])
    pltpu.sync_copy(a3, out3.at[pl.ds(off, C), :])
    pltpu.sync_copy(a4, out4.at[pl.ds(off, C), :])
    return 0

  lax.fori_loop(0, N_CHUNKS, chunk_body, 0)


_sc_gather = functools.partial(
    pl.kernel,
    out_type=tuple(jax.ShapeDtypeStruct((B, 128), jnp.float32) for _ in range(G)),
    mesh=plsc.VectorSubcoreMesh(core_axis_name="c", subcore_axis_name="s"),
    compiler_params=pltpu.CompilerParams(use_tc_tiling_on_sc=False),
    scratch_types=[
        pltpu.VMEM((C * N_DNS,), jnp.int32),
        pltpu.VMEM((C * N_DSEQ,), jnp.int32),
        pltpu.VMEM((C * N_SNS,), jnp.int32),
        pltpu.VMEM((C * N_SSEQ,), jnp.int32),
        pltpu.VMEM((C * N_DNS, E), jnp.float32),
        pltpu.VMEM((C * N_DSEQ, E), jnp.float32),
        pltpu.VMEM((C * N_SNS, E), jnp.float32),
        pltpu.VMEM((C * N_SSEQ, E), jnp.float32),
        pltpu.VMEM((C, 128), jnp.float32),
        pltpu.VMEM((C, 128), jnp.float32),
        pltpu.VMEM((C, 128), jnp.float32),
        pltpu.VMEM((C, 128), jnp.float32),
        pltpu.VMEM((C, 128), jnp.float32),
        pltpu.SemaphoreType.DMA,
    ],
)(_sc_body)


BM = 512


def _tc_body(x0_ref, x1_ref, x2_ref, x3_ref, x4_ref, w_ref, b_ref, out_ref):
  w = w_ref[...]
  acc = jnp.dot(x0_ref[...], w[0:128], preferred_element_type=jnp.float32)
  acc += jnp.dot(x1_ref[...], w[128:256], preferred_element_type=jnp.float32)
  acc += jnp.dot(x2_ref[...], w[256:384], preferred_element_type=jnp.float32)
  acc += jnp.dot(x3_ref[...], w[384:512], preferred_element_type=jnp.float32)
  acc += jnp.dot(x4_ref[...], w[512:640], preferred_element_type=jnp.float32)
  acc += b_ref[0:1, :]
  out_ref[...] = jnp.maximum(acc, 0.0)


def _tc_matmul(xs, wpad, b8):
  grid = (B // BM,)
  return pl.pallas_call(
      _tc_body,
      grid=grid,
      in_specs=[pl.BlockSpec((BM, 128), lambda i: (i, 0)) for _ in range(G)]
      + [
          pl.BlockSpec((640, 128), lambda i: (0, 0)),
          pl.BlockSpec((8, 128), lambda i: (0, 0)),
      ],
      out_specs=pl.BlockSpec((BM, 128), lambda i: (i, 0)),
      out_shape=jax.ShapeDtypeStruct((B, 128), jnp.float32),
  )(*xs, wpad, b8)


def kernel(dense_scalar_ids, dense_seq_ids, sparse_scalar_ids, sparse_seq_ids,
           dense_table, sparse_table, W, b):
  dns_ids = dense_scalar_ids.reshape(-1)
  dseq_ids = dense_seq_ids.reshape(-1)
  sns_ids = sparse_scalar_ids.reshape(-1)
  sseq_ids = sparse_seq_ids.reshape(-1)
  xs = _sc_gather(dense_table, sparse_table,
                  dns_ids, dseq_ids, sns_ids, sseq_ids)
  wpad = jnp.concatenate([W, jnp.zeros((G * 128 - 608, 128), W.dtype)], axis=0)
  b8 = jnp.broadcast_to(b, (8, 128))
  return _tc_matmul(xs, wpad, b8)


# speedup vs baseline: 2.1653x; 1.1840x over previous
import functools

import jax
import jax.numpy as jnp
from jax import lax
from jax.experimental import pallas as pl
from jax.experimental.pallas import tpu as pltpu
from jax.experimental.pallas import tpu_sc as plsc

NC = 2
NS = 16
NW = NC * NS

B = 16384
E = 16
N_DNS = 10
N_DSEQ = 50
N_SNS = 26
N_SSEQ = 20
G = 5

C = 32
B_PER_W = B // NW
N_CHUNKS = B_PER_W // C


def _sc_body(dense_table_flat, sparse_table_flat, dns_ids, dseq_ids, sns_ids, sseq_ids,
             out0, out1, out2, out3, out4,
             dns_idx, dseq_idx, sns_idx, sseq_idx,
             dns_buf, dseq_buf, sns_buf, sseq_buf,
             a0, a1, a2, a3, a4, sem):
  dense_table = dense_table_flat
  sparse_table = sparse_table_flat
  wid = lax.axis_index("s") * NC + lax.axis_index("c")
  base0 = wid * B_PER_W
  zeros = jnp.zeros((16,), jnp.float32)

  def chunk_body(cidx, _):
    base = base0 + cidx * C

    pltpu.sync_copy(dns_ids.at[pl.ds(pl.multiple_of(base * N_DNS, 128), C * N_DNS)], dns_idx)
    pltpu.sync_copy(dseq_ids.at[pl.ds(pl.multiple_of(base * N_DSEQ, 128), C * N_DSEQ)], dseq_idx)
    pltpu.sync_copy(sns_ids.at[pl.ds(pl.multiple_of(base * N_SNS, 128), C * N_SNS)], sns_idx)
    pltpu.sync_copy(sseq_ids.at[pl.ds(pl.multiple_of(base * N_SSEQ, 128), C * N_SSEQ)], sseq_idx)

    def permute(n, idx):
      def pv(t, _):
        v = idx[pl.ds(t * 16, 16)]
        g = ((v & jnp.int32(~(TW - 1)))
             | ((v & jnp.int32(511)) << 3)
             | ((v >> 9) & jnp.int32(7)))
        idx[pl.ds(t * 16, 16)] = g
        return 0
      lax.fori_loop(0, n // 16, pv, 0)

    permute(C * N_DNS, dns_idx)
    permute(C * N_DSEQ, dseq_idx)
    permute(C * N_SNS, sns_idx)
    permute(C * N_SSEQ, sseq_idx)

    def fire(n, table, idx, buf):
      full, rem = n // 128, n % 128
      def go(j, _):
        off = pl.multiple_of(j * 128, 128)
        pltpu.async_copy(table.at[idx.at[pl.ds(off, 128)]],
                         buf.at[pl.ds(off, 128), :], sem)
        return 0
      lax.fori_loop(0, full, go, 0)
      if rem:
        off = full * 128
        pltpu.async_copy(table.at[idx.at[pl.ds(off, rem)]],
                         buf.at[pl.ds(off, rem), :], sem)

    fire(C * N_DNS, dense_table, dns_idx, dns_buf)
    fire(C * N_DSEQ, dense_table, dseq_idx, dseq_buf)
    fire(C * N_SNS, sparse_table, sns_idx, sns_buf)
    fire(C * N_SSEQ, sparse_table, sseq_idx, sseq_buf)

    def drain(n, idx, buf):
      full, rem = n // 128, n % 128
      def dr(j, _):
        pltpu.make_async_copy(dense_table.at[idx.at[pl.ds(0, 128)]],
                              buf.at[pl.ds(0, 128), :], sem).wait()
        return 0
      lax.fori_loop(0, full, dr, 0)
      if rem:
        pltpu.make_async_copy(dense_table.at[idx.at[pl.ds(0, rem)]],
                              buf.at[pl.ds(0, rem), :], sem).wait()

    drain(C * N_DNS, dns_idx, dns_buf)
    drain(C * N_DSEQ, dseq_idx, dseq_buf)
    drain(C * N_SNS, sns_idx, sns_buf)
    drain(C * N_SSEQ, sseq_idx, sseq_buf)

    def row_body(i, _):
      db = i * N_DSEQ
      acc = dseq_buf[db]
      for t in range(1, N_DSEQ):
        acc = acc + dseq_buf[db + t]
      dmean = acc * (1.0 / N_DSEQ)
      sb = i * N_SSEQ
      acc2 = sseq_buf[sb]
      for t in range(1, N_SSEQ):
        acc2 = acc2 + sseq_buf[sb + t]
      smean = acc2 * (1.0 / N_SSEQ)

      dn = i * N_DNS
      sn = i * N_SNS
      for j in range(8):
        a0[i, pl.ds(16 * j, 16)] = dns_buf[dn + j]
      a1[i, pl.ds(0, 16)] = dns_buf[dn + 8]
      a1[i, pl.ds(16, 16)] = dns_buf[dn + 9]
      a1[i, pl.ds(32, 16)] = dmean
      for j in range(5):
        a1[i, pl.ds(48 + 16 * j, 16)] = sns_buf[sn + j]
      for j in range(8):
        a2[i, pl.ds(16 * j, 16)] = sns_buf[sn + 5 + j]
      for j in range(8):
        a3[i, pl.ds(16 * j, 16)] = sns_buf[sn + 13 + j]
      for j in range(5):
        a4[i, pl.ds(16 * j, 16)] = sns_buf[sn + 21 + j]
      a4[i, pl.ds(80, 16)] = smean
      a4[i, pl.ds(96, 16)] = zeros
      a4[i, pl.ds(112, 16)] = zeros
      return 0
    lax.fori_loop(0, C, row_body, 0)

    off = pl.multiple_of(base, C)
    pltpu.sync_copy(a0, out0.at[pl.ds(off, C), :])
    pltpu.sync_copy(a1, out1.at[pl.ds(off, C), :])
    pltpu.sync_copy(a2, out2.at[pl.ds(off, C), :])
    pltpu.sync_copy(a3, out3.at[pl.ds(off, C), :])
    pltpu.sync_copy(a4, out4.at[pl.ds(off, C), :])
    return 0

  lax.fori_loop(0, N_CHUNKS, chunk_body, 0)


_sc_gather = functools.partial(
    pl.kernel,
    out_type=tuple(jax.ShapeDtypeStruct((B, 128), jnp.float32) for _ in range(G)),
    mesh=plsc.VectorSubcoreMesh(core_axis_name="c", subcore_axis_name="s"),
    compiler_params=pltpu.CompilerParams(use_tc_tiling_on_sc=False),
    scratch_types=[
        pltpu.VMEM((C * N_DNS,), jnp.int32),
        pltpu.VMEM((C * N_DSEQ,), jnp.int32),
        pltpu.VMEM((C * N_SNS,), jnp.int32),
        pltpu.VMEM((C * N_SSEQ,), jnp.int32),
        pltpu.VMEM((C * N_DNS, E), jnp.float32),
        pltpu.VMEM((C * N_DSEQ, E), jnp.float32),
        pltpu.VMEM((C * N_SNS, E), jnp.float32),
        pltpu.VMEM((C * N_SSEQ, E), jnp.float32),
        pltpu.VMEM((C, 128), jnp.float32),
        pltpu.VMEM((C, 128), jnp.float32),
        pltpu.VMEM((C, 128), jnp.float32),
        pltpu.VMEM((C, 128), jnp.float32),
        pltpu.VMEM((C, 128), jnp.float32),
        pltpu.SemaphoreType.DMA,
    ],
)(_sc_body)


TW = 4096
V = 1000000
NTB = -(-V // TW)
VP = NTB * TW


def _tr_body(in_ref, out_ref):
  x = in_ref[...]
  for k in range(8):
    xk = x[:, 512 * k:512 * (k + 1)]
    out_ref[:, pl.ds(16 * k, 16)] = xk.T


def _tc_transpose(table_t):
  out = pl.pallas_call(
      _tr_body,
      grid=(NTB,),
      in_specs=[pl.BlockSpec((E, TW), lambda i: (0, i))],
      out_specs=pl.BlockSpec((TW // 8, 128), lambda i: (i, 0)),
      out_shape=jax.ShapeDtypeStruct((VP * E // 128, 128), jnp.float32),
  )(table_t)
  return out.reshape(VP, E)


BM = 512


def _tc_body(x0_ref, x1_ref, x2_ref, x3_ref, x4_ref, w_ref, b_ref, out_ref):
  w = w_ref[...]
  acc = jnp.dot(x0_ref[...], w[0:128], preferred_element_type=jnp.float32)
  acc += jnp.dot(x1_ref[...], w[128:256], preferred_element_type=jnp.float32)
  acc += jnp.dot(x2_ref[...], w[256:384], preferred_element_type=jnp.float32)
  acc += jnp.dot(x3_ref[...], w[384:512], preferred_element_type=jnp.float32)
  acc += jnp.dot(x4_ref[...], w[512:640], preferred_element_type=jnp.float32)
  acc += b_ref[0:1, :]
  out_ref[...] = jnp.maximum(acc, 0.0)


def _tc_matmul(xs, wpad, b8):
  grid = (B // BM,)
  return pl.pallas_call(
      _tc_body,
      grid=grid,
      in_specs=[pl.BlockSpec((BM, 128), lambda i: (i, 0)) for _ in range(G)]
      + [
          pl.BlockSpec((640, 128), lambda i: (0, 0)),
          pl.BlockSpec((8, 128), lambda i: (0, 0)),
      ],
      out_specs=pl.BlockSpec((BM, 128), lambda i: (i, 0)),
      out_shape=jax.ShapeDtypeStruct((B, 128), jnp.float32),
  )(*xs, wpad, b8)


def kernel(dense_scalar_ids, dense_seq_ids, sparse_scalar_ids, sparse_seq_ids,
           dense_table, sparse_table, W, b):
  dns_ids = dense_scalar_ids.reshape(-1)
  dseq_ids = dense_seq_ids.reshape(-1)
  sns_ids = sparse_scalar_ids.reshape(-1)
  sseq_ids = sparse_seq_ids.reshape(-1)
  dt = _tc_transpose(dense_table.T)
  st = _tc_transpose(sparse_table.T)
  xs = _sc_gather(dt, st, dns_ids, dseq_ids, sns_ids, sseq_ids)
  wpad = jnp.concatenate([W, jnp.zeros((G * 128 - 608, 128), W.dtype)], axis=0)
  b8 = jnp.broadcast_to(b, (8, 128))
  return _tc_matmul(xs, wpad, b8)


# speedup vs baseline: 2.9339x; 1.3550x over previous
import functools

import jax
import jax.numpy as jnp
from jax import lax
from jax.experimental import pallas as pl
from jax.experimental.pallas import tpu as pltpu
from jax.experimental.pallas import tpu_sc as plsc

NC = 2
NS = 16
NW = NC * NS

B = 16384
E = 16
N_DNS = 10
N_DSEQ = 50
N_SNS = 26
N_SSEQ = 20
G = 5

C = 32
B_PER_W = B // NW
N_CHUNKS = B_PER_W // C


def _sc_body(dense_table_flat, sparse_table_flat, dns_ids, dseq_ids, sns_ids, sseq_ids,
             out0, out1, out2, out3, out4,
             dns_idx, dseq_idx, sns_idx, sseq_idx,
             dns_buf, dseq_buf, sns_buf, sseq_buf,
             a0, a1, a2, a3, a4, sem):
  dense_table = dense_table_flat
  sparse_table = sparse_table_flat
  wid = lax.axis_index("s") * NC + lax.axis_index("c")
  base0 = wid * B_PER_W
  zeros = jnp.zeros((16,), jnp.float32)

  def chunk_body(cidx, _):
    base = base0 + cidx * C

    pltpu.sync_copy(dns_ids.at[pl.ds(pl.multiple_of(base * N_DNS, 128), C * N_DNS)], dns_idx)
    pltpu.sync_copy(dseq_ids.at[pl.ds(pl.multiple_of(base * N_DSEQ, 128), C * N_DSEQ)], dseq_idx)
    pltpu.sync_copy(sns_ids.at[pl.ds(pl.multiple_of(base * N_SNS, 128), C * N_SNS)], sns_idx)
    pltpu.sync_copy(sseq_ids.at[pl.ds(pl.multiple_of(base * N_SSEQ, 128), C * N_SSEQ)], sseq_idx)

    def permute(n, idx):
      def pv(t, _):
        v = idx[pl.ds(t * 16, 16)]
        g = ((v & jnp.int32(~(TW - 1)))
             | ((v & jnp.int32(511)) << 3)
             | ((v >> 9) & jnp.int32(7)))
        idx[pl.ds(t * 16, 16)] = g
        return 0
      lax.fori_loop(0, n // 16, pv, 0)

    permute(C * N_DNS, dns_idx)
    permute(C * N_DSEQ, dseq_idx)
    permute(C * N_SNS, sns_idx)
    permute(C * N_SSEQ, sseq_idx)

    def fire(n, table, idx, buf):
      full, rem = n // 128, n % 128
      def go(j, _):
        off = pl.multiple_of(j * 128, 128)
        pltpu.async_copy(table.at[idx.at[pl.ds(off, 128)]],
                         buf.at[pl.ds(off, 128), :], sem)
        return 0
      lax.fori_loop(0, full, go, 0)
      if rem:
        off = full * 128
        pltpu.async_copy(table.at[idx.at[pl.ds(off, rem)]],
                         buf.at[pl.ds(off, rem), :], sem)

    fire(C * N_DNS, dense_table, dns_idx, dns_buf)
    fire(C * N_DSEQ, dense_table, dseq_idx, dseq_buf)
    fire(C * N_SNS, sparse_table, sns_idx, sns_buf)
    fire(C * N_SSEQ, sparse_table, sseq_idx, sseq_buf)

    def drain(n, idx, buf):
      full, rem = n // 128, n % 128
      def dr(j, _):
        pltpu.make_async_copy(dense_table.at[idx.at[pl.ds(0, 128)]],
                              buf.at[pl.ds(0, 128), :], sem).wait()
        return 0
      lax.fori_loop(0, full, dr, 0)
      if rem:
        pltpu.make_async_copy(dense_table.at[idx.at[pl.ds(0, rem)]],
                              buf.at[pl.ds(0, rem), :], sem).wait()

    drain(C * N_DNS, dns_idx, dns_buf)
    drain(C * N_DSEQ, dseq_idx, dseq_buf)
    drain(C * N_SNS, sns_idx, sns_buf)
    drain(C * N_SSEQ, sseq_idx, sseq_buf)

    def row_body(i, _):
      db = i * N_DSEQ
      acc = dseq_buf[db]
      for t in range(1, N_DSEQ):
        acc = acc + dseq_buf[db + t]
      dmean = acc * (1.0 / N_DSEQ)
      sb = i * N_SSEQ
      acc2 = sseq_buf[sb]
      for t in range(1, N_SSEQ):
        acc2 = acc2 + sseq_buf[sb + t]
      smean = acc2 * (1.0 / N_SSEQ)

      dn = i * N_DNS
      sn = i * N_SNS
      for j in range(8):
        a0[i, pl.ds(16 * j, 16)] = dns_buf[dn + j]
      a1[i, pl.ds(0, 16)] = dns_buf[dn + 8]
      a1[i, pl.ds(16, 16)] = dns_buf[dn + 9]
      a1[i, pl.ds(32, 16)] = dmean
      for j in range(5):
        a1[i, pl.ds(48 + 16 * j, 16)] = sns_buf[sn + j]
      for j in range(8):
        a2[i, pl.ds(16 * j, 16)] = sns_buf[sn + 5 + j]
      for j in range(8):
        a3[i, pl.ds(16 * j, 16)] = sns_buf[sn + 13 + j]
      for j in range(5):
        a4[i, pl.ds(16 * j, 16)] = sns_buf[sn + 21 + j]
      a4[i, pl.ds(80, 16)] = smean
      a4[i, pl.ds(96, 16)] = zeros
      a4[i, pl.ds(112, 16)] = zeros
      return 0
    lax.fori_loop(0, C, row_body, 0)

    off = pl.multiple_of(base, C)
    pltpu.sync_copy(a0, out0.at[pl.ds(off, C), :])
    pltpu.sync_copy(a1, out1.at[pl.ds(off, C), :])
    pltpu.sync_copy(a2, out2.at[pl.ds(off, C), :])
    pltpu.sync_copy(a3, out3.at[pl.ds(off, C), :])
    pltpu.sync_copy(a4, out4.at[pl.ds(off, C), :])
    return 0

  lax.fori_loop(0, N_CHUNKS, chunk_body, 0)


_sc_gather = functools.partial(
    pl.kernel,
    out_type=tuple(jax.ShapeDtypeStruct((B, 128), jnp.float32) for _ in range(G)),
    mesh=plsc.VectorSubcoreMesh(core_axis_name="c", subcore_axis_name="s"),
    compiler_params=pltpu.CompilerParams(use_tc_tiling_on_sc=False),
    scratch_types=[
        pltpu.VMEM((C * N_DNS,), jnp.int32),
        pltpu.VMEM((C * N_DSEQ,), jnp.int32),
        pltpu.VMEM((C * N_SNS,), jnp.int32),
        pltpu.VMEM((C * N_SSEQ,), jnp.int32),
        pltpu.VMEM((C * N_DNS, E), jnp.float32),
        pltpu.VMEM((C * N_DSEQ, E), jnp.float32),
        pltpu.VMEM((C * N_SNS, E), jnp.float32),
        pltpu.VMEM((C * N_SSEQ, E), jnp.float32),
        pltpu.VMEM((C, 128), jnp.float32),
        pltpu.VMEM((C, 128), jnp.float32),
        pltpu.VMEM((C, 128), jnp.float32),
        pltpu.VMEM((C, 128), jnp.float32),
        pltpu.VMEM((C, 128), jnp.float32),
        pltpu.SemaphoreType.DMA,
    ],
)(_sc_body)


TW = 4096
V = 1000000
NTB = -(-V // TW)
VP = NTB * TW


def _tr_body(in_ref, out_ref):
  x = in_ref[...]
  xs = jnp.concatenate([x[:, 512 * k:512 * (k + 1)] for k in range(8)], axis=0)
  out_ref[...] = xs.T


def _tc_transpose(table_t):
  out = pl.pallas_call(
      _tr_body,
      grid=(NTB,),
      in_specs=[pl.BlockSpec((E, TW), lambda i: (0, i))],
      out_specs=pl.BlockSpec((TW // 8, 128), lambda i: (i, 0)),
      out_shape=jax.ShapeDtypeStruct((VP * E // 128, 128), jnp.float32),
  )(table_t)
  return out.reshape(VP, E)


BM = 512


def _tc_body(x0_ref, x1_ref, x2_ref, x3_ref, x4_ref, w_ref, b_ref, out_ref):
  w = w_ref[...]
  acc = jnp.dot(x0_ref[...], w[0:128], preferred_element_type=jnp.float32)
  acc += jnp.dot(x1_ref[...], w[128:256], preferred_element_type=jnp.float32)
  acc += jnp.dot(x2_ref[...], w[256:384], preferred_element_type=jnp.float32)
  acc += jnp.dot(x3_ref[...], w[384:512], preferred_element_type=jnp.float32)
  acc += jnp.dot(x4_ref[...], w[512:640], preferred_element_type=jnp.float32)
  acc += b_ref[0:1, :]
  out_ref[...] = jnp.maximum(acc, 0.0)


def _tc_matmul(xs, wpad, b8):
  grid = (B // BM,)
  return pl.pallas_call(
      _tc_body,
      grid=grid,
      in_specs=[pl.BlockSpec((BM, 128), lambda i: (i, 0)) for _ in range(G)]
      + [
          pl.BlockSpec((640, 128), lambda i: (0, 0)),
          pl.BlockSpec((8, 128), lambda i: (0, 0)),
      ],
      out_specs=pl.BlockSpec((BM, 128), lambda i: (i, 0)),
      out_shape=jax.ShapeDtypeStruct((B, 128), jnp.float32),
  )(*xs, wpad, b8)


def kernel(dense_scalar_ids, dense_seq_ids, sparse_scalar_ids, sparse_seq_ids,
           dense_table, sparse_table, W, b):
  dns_ids = dense_scalar_ids.reshape(-1)
  dseq_ids = dense_seq_ids.reshape(-1)
  sns_ids = sparse_scalar_ids.reshape(-1)
  sseq_ids = sparse_seq_ids.reshape(-1)
  dt = _tc_transpose(dense_table.T)
  st = _tc_transpose(sparse_table.T)
  xs = _sc_gather(dt, st, dns_ids, dseq_ids, sns_ids, sseq_ids)
  wpad = jnp.concatenate([W, jnp.zeros((G * 128 - 608, 128), W.dtype)], axis=0)
  b8 = jnp.broadcast_to(b, (8, 128))
  return _tc_matmul(xs, wpad, b8)


# speedup vs baseline: 3.5088x; 1.1960x over previous
import functools

import jax
import jax.numpy as jnp
from jax import lax
from jax.experimental import pallas as pl
from jax.experimental.pallas import tpu as pltpu
from jax.experimental.pallas import tpu_sc as plsc

NC = 2
NS = 16
NW = NC * NS

B = 16384
E = 16
N_DNS = 10
N_DSEQ = 50
N_SNS = 26
N_SSEQ = 20
G = 5

C = 16
B_PER_W = B // NW
N_CHUNKS = B_PER_W // C

OFF_DNS = 0
OFF_DSEQ = C * N_DNS
OFF_SNS = OFF_DSEQ + C * N_DSEQ
OFF_SSEQ = OFF_SNS + C * N_SNS
IDS_PER_CHUNK = OFF_SSEQ + C * N_SSEQ


def _streams():
  out = []
  for dense, off, n in ((True, OFF_DNS, C * N_DNS),
                        (True, OFF_DSEQ, C * N_DSEQ),
                        (False, OFF_SNS, C * N_SNS),
                        (False, OFF_SSEQ, C * N_SSEQ)):
    p = 0
    while p < n:
      sz = min(128, n - p)
      out.append((dense, off + p, sz))
      p += sz
  return out


STREAMS = _streams()

TW = 4096
V = 1000000
NTB = -(-V // TW)
VP = NTB * TW


def _sc_body(dense_table, sparse_table, ids_cat,
             out0, out1, out2, out3, out4,
             idx0, idx1, buf0, buf1,
             a00, a01, a02, a03, a04,
             a10, a11, a12, a13, a14,
             gsem0, gsem1, isem, wsem0, wsem1):
  wid = lax.axis_index("s") * NC + lax.axis_index("c")
  base_chunk = wid * N_CHUNKS
  zeros = jnp.zeros((16,), jnp.float32)
  aset0 = (a00, a01, a02, a03, a04)
  aset1 = (a10, a11, a12, a13, a14)
  outs = (out0, out1, out2, out3, out4)

  def stage(c, idx, sem):
    off = pl.multiple_of((base_chunk + c) * IDS_PER_CHUNK, 8)
    return pltpu.async_copy(ids_cat.at[pl.ds(off, IDS_PER_CHUNK)], idx, sem)

  def wait_ids(idx):
    pltpu.make_async_copy(ids_cat.at[pl.ds(0, IDS_PER_CHUNK)], idx, isem).wait()

  def permute(idx):
    def pv(t, _):
      v = idx[pl.ds(t * 16, 16)]
      g = ((v & jnp.int32(~(TW - 1)))
           | ((v & jnp.int32(511)) << 3)
           | ((v >> 9) & jnp.int32(7)))
      idx[pl.ds(t * 16, 16)] = g
      return 0
    lax.fori_loop(0, IDS_PER_CHUNK // 16, pv, 0)

  def fire(idx, buf, gsem):
    for dense, off, sz in STREAMS:
      table = dense_table if dense else sparse_table
      pltpu.async_copy(table.at[idx.at[pl.ds(off, sz)]],
                       buf.at[pl.ds(off, sz), :], gsem)

  def drain(gsem):
    for _, _, sz in STREAMS:
      pltpu.make_async_copy(dense_table.at[idx0.at[pl.ds(0, sz)]],
                            buf0.at[pl.ds(0, sz), :], gsem).wait()

  def drain_writes(aset, wsem):
    for k in range(G):
      pltpu.make_async_copy(aset[k], outs[k].at[pl.ds(0, C), :], wsem).wait()

  def process(c, buf, aset, wsem):
    a0, a1, a2, a3, a4 = aset

    def row_body(i, _):
      db = OFF_DSEQ + i * N_DSEQ
      acc = buf[db]
      for t in range(1, N_DSEQ):
        acc = acc + buf[db + t]
      dmean = acc * (1.0 / N_DSEQ)
      sb = OFF_SSEQ + i * N_SSEQ
      acc2 = buf[sb]
      for t in range(1, N_SSEQ):
        acc2 = acc2 + buf[sb + t]
      smean = acc2 * (1.0 / N_SSEQ)

      dn = OFF_DNS + i * N_DNS
      sn = OFF_SNS + i * N_SNS
      for j in range(8):
        a0[i, pl.ds(16 * j, 16)] = buf[dn + j]
      a1[i, pl.ds(0, 16)] = buf[dn + 8]
      a1[i, pl.ds(16, 16)] = buf[dn + 9]
      a1[i, pl.ds(32, 16)] = dmean
      for j in range(5):
        a1[i, pl.ds(48 + 16 * j, 16)] = buf[sn + j]
      for j in range(8):
        a2[i, pl.ds(16 * j, 16)] = buf[sn + 5 + j]
      for j in range(8):
        a3[i, pl.ds(16 * j, 16)] = buf[sn + 13 + j]
      for j in range(5):
        a4[i, pl.ds(16 * j, 16)] = buf[sn + 21 + j]
      a4[i, pl.ds(80, 16)] = smean
      a4[i, pl.ds(96, 16)] = zeros
      a4[i, pl.ds(112, 16)] = zeros
      return 0
    lax.fori_loop(0, C, row_body, 0)

    off = pl.multiple_of(wid * B_PER_W + c * C, C)
    for k in range(G):
      pltpu.async_copy(aset[k], outs[k].at[pl.ds(off, C), :], wsem)

  pltpu.sync_copy(
      ids_cat.at[pl.ds(pl.multiple_of(base_chunk * IDS_PER_CHUNK, 8),
                       IDS_PER_CHUNK)], idx0)
  permute(idx0)
  fire(idx0, buf0, gsem0)
  stage(1, idx1, isem)

  def pair_body(h, _):
    ca = 2 * h

    wait_ids(idx1)
    permute(idx1)
    fire(idx1, buf1, gsem1)
    drain(gsem0)

    @pl.when(h < (N_CHUNKS // 2) - 1)
    def _():
      stage(ca + 2, idx0, isem)

    @pl.when(h > 0)
    def _():
      drain_writes(aset0, wsem0)
    process(ca, buf0, aset0, wsem0)

    @pl.when(h < (N_CHUNKS // 2) - 1)
    def _():
      wait_ids(idx0)
      permute(idx0)
      fire(idx0, buf0, gsem0)
    drain(gsem1)

    @pl.when(h < (N_CHUNKS // 2) - 1)
    def _():
      stage(ca + 3, idx1, isem)

    @pl.when(h > 0)
    def _():
      drain_writes(aset1, wsem1)
    process(ca + 1, buf1, aset1, wsem1)
    return 0

  lax.fori_loop(0, N_CHUNKS // 2, pair_body, 0)
  drain_writes(aset0, wsem0)
  drain_writes(aset1, wsem1)


_sc_gather = functools.partial(
    pl.kernel,
    out_type=tuple(jax.ShapeDtypeStruct((B, 128), jnp.float32) for _ in range(G)),
    mesh=plsc.VectorSubcoreMesh(core_axis_name="c", subcore_axis_name="s"),
    compiler_params=pltpu.CompilerParams(use_tc_tiling_on_sc=False),
    scratch_types=[
        pltpu.VMEM((IDS_PER_CHUNK,), jnp.int32),
        pltpu.VMEM((IDS_PER_CHUNK,), jnp.int32),
        pltpu.VMEM((IDS_PER_CHUNK, E), jnp.float32),
        pltpu.VMEM((IDS_PER_CHUNK, E), jnp.float32),
    ]
    + [pltpu.VMEM((C, 128), jnp.float32) for _ in range(2 * G)]
    + [pltpu.SemaphoreType.DMA] * 5,
)(_sc_body)


def _tr_body(in_ref, out_ref):
  x = in_ref[...]
  xs = jnp.concatenate([x[:, 512 * k:512 * (k + 1)] for k in range(8)], axis=0)
  out_ref[...] = xs.T


def _tc_transpose(table_t):
  out = pl.pallas_call(
      _tr_body,
      grid=(NTB,),
      in_specs=[pl.BlockSpec((E, TW), lambda i: (0, i))],
      out_specs=pl.BlockSpec((TW // 8, 128), lambda i: (i, 0)),
      out_shape=jax.ShapeDtypeStruct((VP * E // 128, 128), jnp.float32),
  )(table_t)
  return out.reshape(VP, E)


BM = 512


def _tc_body(x0_ref, x1_ref, x2_ref, x3_ref, x4_ref, w_ref, b_ref, out_ref):
  w = w_ref[...]
  acc = jnp.dot(x0_ref[...], w[0:128], preferred_element_type=jnp.float32)
  acc += jnp.dot(x1_ref[...], w[128:256], preferred_element_type=jnp.float32)
  acc += jnp.dot(x2_ref[...], w[256:384], preferred_element_type=jnp.float32)
  acc += jnp.dot(x3_ref[...], w[384:512], preferred_element_type=jnp.float32)
  acc += jnp.dot(x4_ref[...], w[512:640], preferred_element_type=jnp.float32)
  acc += b_ref[0:1, :]
  out_ref[...] = jnp.maximum(acc, 0.0)


def _tc_matmul(xs, wpad, b8):
  grid = (B // BM,)
  return pl.pallas_call(
      _tc_body,
      grid=grid,
      in_specs=[pl.BlockSpec((BM, 128), lambda i: (i, 0)) for _ in range(G)]
      + [
          pl.BlockSpec((G * 128, 128), lambda i: (0, 0)),
          pl.BlockSpec((8, 128), lambda i: (0, 0)),
      ],
      out_specs=pl.BlockSpec((BM, 128), lambda i: (i, 0)),
      out_shape=jax.ShapeDtypeStruct((B, 128), jnp.float32),
  )(*xs, wpad, b8)


def kernel(dense_scalar_ids, dense_seq_ids, sparse_scalar_ids, sparse_seq_ids,
           dense_table, sparse_table, W, b):
  nch = B // C
  ids_cat = jnp.concatenate(
      [dense_scalar_ids.reshape(nch, C * N_DNS),
       dense_seq_ids.reshape(nch, C * N_DSEQ),
       sparse_scalar_ids.reshape(nch, C * N_SNS),
       sparse_seq_ids.reshape(nch, C * N_SSEQ)], axis=1).reshape(-1)
  dt = _tc_transpose(dense_table.T)
  st = _tc_transpose(sparse_table.T)
  xs = _sc_gather(dt, st, ids_cat)
  wpad = jnp.concatenate([W, jnp.zeros((G * 128 - 608, 128), W.dtype)], axis=0)
  b8 = jnp.broadcast_to(b, (8, 128))
  return _tc_matmul(xs, wpad, b8)
